# Initial kernel scaffold; baseline (speedup 1.0000x reference)
import jax
import jax.numpy as jnp
from jax.experimental import pallas as pl


def kernel(x, edge_index, W_embed, b_embed, W_lin1, b_lin1, W_gat0, attn_l0, attn_r0, b_gat0, W_gat1, attn_l1, attn_r1, b_gat1, W_ih0, W_hh0, b_ih0, b_hh0, W_ih1, W_hh1, b_ih1, b_hh1, W_pred, b_pred):
    raise NotImplementedError("write your pallas kernel here")


# speedup vs baseline: 21.6861x; 21.6861x over previous
import functools

import jax
import jax.numpy as jnp
from jax import lax
from jax.experimental import pallas as pl
from jax.experimental.pallas import tpu as pltpu
from jax.experimental.pallas import tpu_sc as plsc

N_PAD = 10240
HID = 16
NT = 16
NPT = N_PAD // NT
CH = 128


def _embed_body(x_ref, we_ref, be_ref, wl_ref, bl_ref, o_ref):
    h = jnp.dot(x_ref[...], we_ref[...], preferred_element_type=jnp.float32)
    h = h + be_ref[...]
    o_ref[...] = (
        jnp.dot(h, wl_ref[...], preferred_element_type=jnp.float32) + bl_ref[...]
    )


def _tc_embed(x_p, W_embed, b_embed, W_lin1, b_lin1):
    return pl.pallas_call(
        _embed_body,
        out_shape=jax.ShapeDtypeStruct((N_PAD, HID), jnp.float32),
    )(x_p, W_embed, b_embed.reshape(1, -1), W_lin1, b_lin1.reshape(1, -1))


def _prep_body(h_ref, b_ref, W_ref, al_ref, ar_ref, hp_ref, eler_ref):
    h = h_ref[...] + b_ref[...]
    hp = jnp.dot(h, W_ref[...], preferred_element_type=jnp.float32)
    hp_ref[...] = hp
    el = lax.dot_general(al_ref[...], hp, (((1,), (1,)), ((), ())),
                         preferred_element_type=jnp.float32)
    er = lax.dot_general(ar_ref[...], hp, (((1,), (1,)), ((), ())),
                         preferred_element_type=jnp.float32)
    zm = jnp.max(el) + jnp.max(er)
    m_sc = jnp.where(zm >= 0, zm, zm * 0.2)
    eler_ref[...] = jnp.concatenate(
        [el, er, jnp.full((1, el.shape[1]), m_sc, jnp.float32),
         jnp.zeros((5, el.shape[1]), jnp.float32)], axis=0)


def _tc_prep(h, badd, W, al, ar):
    return pl.pallas_call(
        _prep_body,
        out_shape=[
            jax.ShapeDtypeStruct((N_PAD, HID), jnp.float32),
            jax.ShapeDtypeStruct((8, N_PAD), jnp.float32),
        ],
    )(h, badd.reshape(1, -1), W, al.reshape(1, -1), ar.reshape(1, -1))


def _sc_gat_body(src_hbm, dst_hbm, hp_hbm, eler_hbm, out_hbm,
                 el_t, er_t, den_t, w_t, m_t, srcb, dstb, rows, zb1, zbuf,
                 hp_s, den_s, out_s, sem, *, ept):
    nch = ept // CH
    wid = lax.axis_index("s")
    row0 = wid * NPT
    ebase = wid * ept

    pltpu.sync_copy(hp_hbm.at[pl.ds(row0, NPT), :], hp_s.at[pl.ds(row0, NPT), :])
    pltpu.sync_copy(eler_hbm.at[0], el_t)
    pltpu.sync_copy(eler_hbm.at[1], er_t)
    pltpu.sync_copy(eler_hbm.at[2, pl.ds(0, 16)], m_t)

    def _z2(i, _):
        zbuf[i, :] = jnp.zeros((HID,), jnp.float32)
        return 0
    lax.fori_loop(0, NPT, _z2, 0)

    def _z1(i, _):
        zb1[pl.ds(i * 16, 16)] = jnp.zeros((16,), jnp.float32)
        return 0
    lax.fori_loop(0, NPT // 16, _z1, 0)

    pltpu.sync_copy(zbuf, out_s.at[pl.ds(row0, NPT), :])
    pltpu.sync_copy(zb1, den_s.at[pl.ds(row0, NPT)])

    m_vec = m_t[...]

    plsc.subcore_barrier()

    def _p1(c, _):
        off = ebase + c * CH
        pltpu.sync_copy(src_hbm.at[pl.ds(off, CH)], srcb)
        pltpu.sync_copy(dst_hbm.at[pl.ds(off, CH)], dstb)
        for j in range(CH // 16):
            s16 = srcb[pl.ds(j * 16, 16)]
            d16 = dstb[pl.ds(j * 16, 16)]
            zz = plsc.load_gather(el_t, [s16]) + plsc.load_gather(er_t, [d16])
            e = jnp.where(zz >= 0, zz, zz * 0.2)
            w_t[pl.ds(c * CH + j * 16, 16)] = jnp.exp(e - m_vec)
        pltpu.sync_copy(w_t.at[pl.ds(c * CH, CH)], den_s.at[dstb], add=True)
        return 0
    lax.fori_loop(0, nch, _p1, 0)

    plsc.subcore_barrier()

    pltpu.sync_copy(den_s, den_t)

    def _p2(c, _):
        off = ebase + c * CH
        pltpu.sync_copy(src_hbm.at[pl.ds(off, CH)], srcb)
        pltpu.sync_copy(dst_hbm.at[pl.ds(off, CH)], dstb)
        pltpu.async_copy(hp_s.at[srcb], rows, sem).wait()
        for j in range(CH // 16):
            d16 = dstb[pl.ds(j * 16, 16)]
            den = plsc.load_gather(den_t, [d16])
            w16 = w_t[pl.ds(c * CH + j * 16, 16)]
            a16 = w16 / (den + 1e-9)
            for k2 in range(16):
                jj = j * 16 + k2
                rows[jj, :] = rows[jj, :] * a16[k2]
        pltpu.sync_copy(rows, out_s.at[dstb], add=True)
        return 0
    lax.fori_loop(0, nch, _p2, 0)

    plsc.subcore_barrier()

    pltpu.sync_copy(out_s.at[pl.ds(row0, NPT), :], out_hbm.at[pl.ds(row0, NPT), :])


def _make_sc_gat(e_pad):
    ept = e_pad // NT
    mesh = plsc.VectorSubcoreMesh(core_axis_name="c", subcore_axis_name="s",
                                  num_cores=1)
    return pl.kernel(
        functools.partial(_sc_gat_body, ept=ept),
        out_type=jax.ShapeDtypeStruct((N_PAD, HID), jnp.float32),
        mesh=mesh,
        compiler_params=pltpu.CompilerParams(
            needs_layout_passes=False, use_tc_tiling_on_sc=False),
        scratch_types=[
            pltpu.VMEM((N_PAD,), jnp.float32),
            pltpu.VMEM((N_PAD,), jnp.float32),
            pltpu.VMEM((N_PAD,), jnp.float32),
            pltpu.VMEM((ept,), jnp.float32),
            pltpu.VMEM((16,), jnp.float32),
            pltpu.VMEM((CH,), jnp.int32),
            pltpu.VMEM((CH,), jnp.int32),
            pltpu.VMEM((CH, HID), jnp.float32),
            pltpu.VMEM((NPT,), jnp.float32),
            pltpu.VMEM((NPT, HID), jnp.float32),
            pltpu.VMEM_SHARED((N_PAD, HID), jnp.float32),
            pltpu.VMEM_SHARED((N_PAD,), jnp.float32),
            pltpu.VMEM_SHARED((N_PAD, HID), jnp.float32),
            pltpu.SemaphoreType.DMA,
        ],
    )


def _lstm_body(t0_ref, t1_ref, h0_ref, b0_ref, b1_ref,
               wih0_ref, bb0_ref, wih1_ref, whh1_ref, bb1_ref,
               wp_ref, bp_ref, o_ref):
    xx = h0_ref[...]
    in0 = jnp.concatenate([t0_ref[...] + b0_ref[...], xx], axis=-1)
    g = jnp.dot(in0, wih0_ref[...], preferred_element_type=jnp.float32) + bb0_ref[...]
    i, f, gg, o = jnp.split(g, 4, axis=-1)
    i = jax.nn.sigmoid(i)
    gg = jnp.tanh(gg)
    o = jax.nn.sigmoid(o)
    c1 = i * gg
    x1 = o * jnp.tanh(c1)
    in1 = jnp.concatenate([t1_ref[...] + b1_ref[...], x1], axis=-1)
    g = (jnp.dot(in1, wih1_ref[...], preferred_element_type=jnp.float32)
         + jnp.dot(x1, whh1_ref[...], preferred_element_type=jnp.float32)
         + bb1_ref[...])
    i, f, gg, o = jnp.split(g, 4, axis=-1)
    i = jax.nn.sigmoid(i)
    f = jax.nn.sigmoid(f)
    gg = jnp.tanh(gg)
    o = jax.nn.sigmoid(o)
    c2 = f * c1 + i * gg
    x2 = o * jnp.tanh(c2)
    y = jnp.dot(x2, wp_ref[...], preferred_element_type=jnp.float32) + bp_ref[...]
    o_ref[...] = jax.nn.sigmoid(y)


def _tc_lstm(t0, b0, t1, b1, h0, W_ih0, b_ih0, b_hh0,
             W_ih1, W_hh1, b_ih1, b_hh1, W_pred, b_pred):
    wp128 = jnp.zeros((HID, 128), jnp.float32).at[:, 0].set(W_pred[:, 0])
    bp128 = jnp.zeros((1, 128), jnp.float32).at[0, 0].set(b_pred[0])
    return pl.pallas_call(
        _lstm_body,
        out_shape=jax.ShapeDtypeStruct((N_PAD, 128), jnp.float32),
    )(t0, t1, h0, b0.reshape(1, -1), b1.reshape(1, -1),
      W_ih0.T, (b_ih0 + b_hh0).reshape(1, -1),
      W_ih1.T, W_hh1.T, (b_ih1 + b_hh1).reshape(1, -1),
      wp128, bp128)


def kernel(x, edge_index, W_embed, b_embed, W_lin1, b_lin1,
           W_gat0, attn_l0, attn_r0, b_gat0,
           W_gat1, attn_l1, attn_r1, b_gat1,
           W_ih0, W_hh0, b_ih0, b_hh0,
           W_ih1, W_hh1, b_ih1, b_hh1,
           W_pred, b_pred):
    n, _ = x.shape
    e = edge_index.shape[1]
    e_pad = ((e + NT * CH - 1) // (NT * CH)) * (NT * CH)

    x_p = jnp.concatenate(
        [x, jnp.zeros((N_PAD - n, x.shape[1]), jnp.float32)], axis=0)
    pad_idx = jnp.full((e_pad - e,), N_PAD - 1, jnp.int32)
    src_p = jnp.concatenate([edge_index[0], pad_idx])
    dst_p = jnp.concatenate([edge_index[1], pad_idx])

    sc_gat = _make_sc_gat(e_pad)

    h0 = _tc_embed(x_p, W_embed, b_embed, W_lin1, b_lin1)

    zb = jnp.zeros((HID,), jnp.float32)

    def gat(h, badd, W, al, ar):
        hp, eler = _tc_prep(h, badd, W, al, ar)
        return sc_gat(src_p, dst_p, hp, eler)

    t0 = gat(h0, zb, W_gat0, attn_l0, attn_r0)
    t0 = gat(t0, b_gat0, W_gat0, attn_l0, attn_r0)
    t1 = gat(h0, zb, W_gat1, attn_l1, attn_r1)
    t1 = gat(t1, b_gat1, W_gat1, attn_l1, attn_r1)

    y = _tc_lstm(t0, b_gat0, t1, b_gat1, h0,
                 W_ih0, b_ih0, b_hh0, W_ih1, W_hh1, b_ih1, b_hh1,
                 W_pred, b_pred)
    return y[:n, :1]


# speedup vs baseline: 47.3218x; 2.1821x over previous
import jax
import jax.numpy as jnp
from jax import lax
from jax.experimental import pallas as pl
from jax.experimental.pallas import tpu as pltpu
from jax.experimental.pallas import tpu_sc as plsc

N_PAD = 10240
HID = 16
NC = 2
NS = 16
NPT = N_PAD // NS
CH = 128
EPS = 1e-9


def _embed_body(x_ref, we_ref, be_ref, wl_ref, bl_ref, o_ref):
    h = jnp.dot(x_ref[...], we_ref[...], preferred_element_type=jnp.float32)
    h = h + be_ref[...]
    o_ref[...] = (
        jnp.dot(h, wl_ref[...], preferred_element_type=jnp.float32) + bl_ref[...]
    )


def _tc_embed(x_p, W_embed, b_embed, W_lin1, b_lin1):
    return pl.pallas_call(
        _embed_body,
        out_shape=jax.ShapeDtypeStruct((N_PAD, HID), jnp.float32),
    )(x_p, W_embed, b_embed.reshape(1, -1), W_lin1, b_lin1.reshape(1, -1))


def _prep_body(s0_ref, s1_ref, d0_ref, d1_ref, b_ref, W_ref, al_ref, ar_ref,
               hp_ref, eler_ref):
    h = (s0_ref[...] + s1_ref[...]) / (d0_ref[...] + d1_ref[...] + EPS)
    h = h + b_ref[...]
    hp = jnp.dot(h, W_ref[...], preferred_element_type=jnp.float32)
    hp_ref[...] = hp
    el = lax.dot_general(al_ref[...], hp, (((1,), (1,)), ((), ())),
                         preferred_element_type=jnp.float32)
    er = lax.dot_general(ar_ref[...], hp, (((1,), (1,)), ((), ())),
                         preferred_element_type=jnp.float32)
    zm = jnp.max(el) + jnp.max(er)
    m_sc = jnp.where(zm >= 0, zm, zm * 0.2)
    eler_ref[...] = jnp.concatenate(
        [el, er, jnp.full((1, el.shape[1]), m_sc, jnp.float32),
         jnp.zeros((5, el.shape[1]), jnp.float32)], axis=0)


def _tc_prep(s0, s1, d0, d1, badd, W, al, ar):
    return pl.pallas_call(
        _prep_body,
        out_shape=[
            jax.ShapeDtypeStruct((N_PAD, HID), jnp.float32),
            jax.ShapeDtypeStruct((8, N_PAD), jnp.float32),
        ],
    )(s0, s1, d0, d1, badd.reshape(1, -1), W,
      al.reshape(1, -1), ar.reshape(1, -1))


def _sc_gat_body(src_hbm, dst_hbm, hp_hbm, eler_hbm, s_hbm, den_hbm,
                 el_t, er_t, m_t, srcb, dstb, wbuf, rows,
                 out_s, den_s, sem, *, ept):
    nch = ept // CH
    cid = lax.axis_index("c")
    sid = lax.axis_index("s")
    gwid = sid * NC + cid
    row0 = sid * NPT
    ebase = gwid * ept

    pltpu.sync_copy(eler_hbm.at[0], el_t)
    pltpu.sync_copy(eler_hbm.at[1], er_t)
    pltpu.sync_copy(eler_hbm.at[2, pl.ds(0, 16)], m_t)

    def _zr(i, _):
        rows[i, :] = jnp.zeros((HID,), jnp.float32)
        return 0
    lax.fori_loop(0, CH, _zr, 0)

    def _zw(i, _):
        wbuf[pl.ds(i * 16, 16)] = jnp.zeros((16,), jnp.float32)
        return 0
    lax.fori_loop(0, CH // 16, _zw, 0)

    for k in range(NPT // CH):
        pltpu.sync_copy(rows, out_s.at[pl.ds(row0 + k * CH, CH), :])
    for k in range(NPT // CH):
        pltpu.sync_copy(wbuf, den_s.at[pl.ds(row0 + k * CH, CH)])

    m_vec = m_t[...]

    plsc.subcore_barrier()

    def _pass(c, _):
        off = ebase + c * CH
        pltpu.sync_copy(src_hbm.at[pl.ds(off, CH)], srcb)
        pltpu.sync_copy(dst_hbm.at[pl.ds(off, CH)], dstb)
        gat = pltpu.async_copy(hp_hbm.at[srcb], rows, sem)
        for j in range(CH // 16):
            s16 = srcb[pl.ds(j * 16, 16)]
            d16 = dstb[pl.ds(j * 16, 16)]
            zz = plsc.load_gather(el_t, [s16]) + plsc.load_gather(er_t, [d16])
            e = jnp.where(zz >= 0, zz, zz * 0.2)
            wbuf[pl.ds(j * 16, 16)] = jnp.exp(e - m_vec)
        pltpu.sync_copy(wbuf, den_s.at[dstb], add=True)
        gat.wait()
        for j in range(CH // 16):
            w16 = wbuf[pl.ds(j * 16, 16)]
            for k2 in range(16):
                jj = j * 16 + k2
                rows[jj, :] = rows[jj, :] * w16[k2]
        pltpu.sync_copy(rows, out_s.at[dstb], add=True)
        return 0
    lax.fori_loop(0, nch, _pass, 0)

    plsc.subcore_barrier()

    pltpu.sync_copy(out_s.at[pl.ds(row0, NPT), :],
                    s_hbm.at[cid, pl.ds(row0, NPT), :])
    pltpu.sync_copy(den_s.at[pl.ds(row0, NPT)],
                    den_hbm.at[cid, pl.ds(row0, NPT)])


def _make_sc_gat(e_pad):
    ept = e_pad // (NC * NS)
    mesh = plsc.VectorSubcoreMesh(core_axis_name="c", subcore_axis_name="s",
                                  num_cores=NC)
    return pl.kernel(
        lambda *args: _sc_gat_body(*args, ept=ept),
        out_type=[
            jax.ShapeDtypeStruct((NC, N_PAD, HID), jnp.float32),
            jax.ShapeDtypeStruct((NC, N_PAD), jnp.float32),
        ],
        mesh=mesh,
        compiler_params=pltpu.CompilerParams(
            needs_layout_passes=False, use_tc_tiling_on_sc=False),
        scratch_types=[
            pltpu.VMEM((N_PAD,), jnp.float32),
            pltpu.VMEM((N_PAD,), jnp.float32),
            pltpu.VMEM((16,), jnp.float32),
            pltpu.VMEM((CH,), jnp.int32),
            pltpu.VMEM((CH,), jnp.int32),
            pltpu.VMEM((CH,), jnp.float32),
            pltpu.VMEM((CH, HID), jnp.float32),
            pltpu.VMEM_SHARED((N_PAD, HID), jnp.float32),
            pltpu.VMEM_SHARED((N_PAD,), jnp.float32),
            pltpu.SemaphoreType.DMA,
        ],
    )


def _lstm_body(sa0_ref, sa1_ref, da0_ref, da1_ref,
               sb0_ref, sb1_ref, db0_ref, db1_ref,
               h0_ref, b0_ref, b1_ref,
               wih0_ref, bb0_ref, wih1_ref, whh1_ref, bb1_ref,
               wp_ref, bp_ref, o_ref):
    t0 = (sa0_ref[...] + sa1_ref[...]) / (da0_ref[...] + da1_ref[...] + EPS)
    t1 = (sb0_ref[...] + sb1_ref[...]) / (db0_ref[...] + db1_ref[...] + EPS)
    xx = h0_ref[...]
    in0 = jnp.concatenate([t0 + b0_ref[...], xx], axis=-1)
    g = jnp.dot(in0, wih0_ref[...], preferred_element_type=jnp.float32) + bb0_ref[...]
    i, f, gg, o = jnp.split(g, 4, axis=-1)
    i = jax.nn.sigmoid(i)
    gg = jnp.tanh(gg)
    o = jax.nn.sigmoid(o)
    c1 = i * gg
    x1 = o * jnp.tanh(c1)
    in1 = jnp.concatenate([t1 + b1_ref[...], x1], axis=-1)
    g = (jnp.dot(in1, wih1_ref[...], preferred_element_type=jnp.float32)
         + jnp.dot(x1, whh1_ref[...], preferred_element_type=jnp.float32)
         + bb1_ref[...])
    i, f, gg, o = jnp.split(g, 4, axis=-1)
    i = jax.nn.sigmoid(i)
    f = jax.nn.sigmoid(f)
    gg = jnp.tanh(gg)
    o = jax.nn.sigmoid(o)
    c2 = f * c1 + i * gg
    x2 = o * jnp.tanh(c2)
    y = jnp.dot(x2, wp_ref[...], preferred_element_type=jnp.float32) + bp_ref[...]
    o_ref[...] = jax.nn.sigmoid(y)


def _tc_lstm(t0p, b0, t1p, b1, h0, W_ih0, b_ih0, b_hh0,
             W_ih1, W_hh1, b_ih1, b_hh1, W_pred, b_pred):
    wp128 = jnp.zeros((HID, 128), jnp.float32).at[:, 0].set(W_pred[:, 0])
    bp128 = jnp.zeros((1, 128), jnp.float32).at[0, 0].set(b_pred[0])
    sa0, sa1, da0, da1 = t0p
    sb0, sb1, db0, db1 = t1p
    blk = 2048
    node = lambda w: pl.BlockSpec((blk, w), lambda i: (i, 0))
    full = lambda a: pl.BlockSpec(a.shape, lambda i: (0,) * a.ndim)
    args = (sa0, sa1, da0, da1, sb0, sb1, db0, db1, h0,
            b0.reshape(1, -1), b1.reshape(1, -1),
            W_ih0.T, (b_ih0 + b_hh0).reshape(1, -1),
            W_ih1.T, W_hh1.T, (b_ih1 + b_hh1).reshape(1, -1),
            wp128, bp128)
    specs = [node(16), node(16), node(1), node(1),
             node(16), node(16), node(1), node(1), node(16)]
    specs += [full(a) for a in args[9:]]
    return pl.pallas_call(
        _lstm_body,
        grid=(N_PAD // blk,),
        in_specs=specs,
        out_specs=pl.BlockSpec((blk, 128), lambda i: (i, 0)),
        out_shape=jax.ShapeDtypeStruct((N_PAD, 128), jnp.float32),
    )(*args)


def kernel(x, edge_index, W_embed, b_embed, W_lin1, b_lin1,
           W_gat0, attn_l0, attn_r0, b_gat0,
           W_gat1, attn_l1, attn_r1, b_gat1,
           W_ih0, W_hh0, b_ih0, b_hh0,
           W_ih1, W_hh1, b_ih1, b_hh1,
           W_pred, b_pred):
    n, _ = x.shape
    e = edge_index.shape[1]
    grp = NC * NS * CH
    e_pad = ((e + grp - 1) // grp) * grp

    x_p = jnp.concatenate(
        [x, jnp.zeros((N_PAD - n, x.shape[1]), jnp.float32)], axis=0)
    pad_idx = jnp.full((e_pad - e,), N_PAD - 1, jnp.int32)
    src_p = jnp.concatenate([edge_index[0], pad_idx])
    dst_p = jnp.concatenate([edge_index[1], pad_idx])

    sc_gat = _make_sc_gat(e_pad)

    h0 = _tc_embed(x_p, W_embed, b_embed, W_lin1, b_lin1)

    zb = jnp.zeros((HID,), jnp.float32)
    zs = jnp.zeros((N_PAD, HID), jnp.float32)
    zd = jnp.zeros((N_PAD, 1), jnp.float32)
    od = jnp.ones((N_PAD, 1), jnp.float32)
    h0_parts = (h0, zs, od, zd)

    def gat(parts, badd, W, al, ar):
        hp, eler = _tc_prep(*parts, badd, W, al, ar)
        s, den = sc_gat(src_p, dst_p, hp, eler)
        return (s[0], s[1], den[0].reshape(N_PAD, 1), den[1].reshape(N_PAD, 1))

    t0p = gat(h0_parts, zb, W_gat0, attn_l0, attn_r0)
    t0p = gat(t0p, b_gat0, W_gat0, attn_l0, attn_r0)
    t1p = gat(h0_parts, zb, W_gat1, attn_l1, attn_r1)
    t1p = gat(t1p, b_gat1, W_gat1, attn_l1, attn_r1)

    y = _tc_lstm(t0p, b_gat0, t1p, b_gat1, h0,
                 W_ih0, b_ih0, b_hh0, W_ih1, W_hh1, b_ih1, b_hh1,
                 W_pred, b_pred)
    return y[:n, :1]


# speedup vs baseline: 59.6655x; 1.2608x over previous
import jax
import jax.numpy as jnp
from jax import lax
from jax.experimental import pallas as pl
from jax.experimental.pallas import tpu as pltpu
from jax.experimental.pallas import tpu_sc as plsc

N_PAD = 10240
HID = 16
NC = 2
NS = 16
NPT = N_PAD // NS
CH = 128
EPS = 1e-9


def _embed_body(x_ref, we_ref, be_ref, wl_ref, bl_ref, o_ref):
    h = jnp.dot(x_ref[...], we_ref[...], preferred_element_type=jnp.float32)
    h = h + be_ref[...]
    o_ref[...] = (
        jnp.dot(h, wl_ref[...], preferred_element_type=jnp.float32) + bl_ref[...]
    )


def _tc_embed(x_p, W_embed, b_embed, W_lin1, b_lin1):
    return pl.pallas_call(
        _embed_body,
        out_shape=jax.ShapeDtypeStruct((N_PAD, HID), jnp.float32),
    )(x_p, W_embed, b_embed.reshape(1, -1), W_lin1, b_lin1.reshape(1, -1))


def _prep_body(s0_ref, s1_ref, d0_ref, d1_ref, b_ref, W_ref, al_ref, ar_ref,
               hp_ref, eler_ref):
    h = (s0_ref[...] + s1_ref[...]) / (d0_ref[...] + d1_ref[...] + EPS)
    h = h + b_ref[...]
    hp = jnp.dot(h, W_ref[...], preferred_element_type=jnp.float32)
    hp_ref[...] = hp
    el = lax.dot_general(al_ref[...], hp, (((1,), (1,)), ((), ())),
                         preferred_element_type=jnp.float32)
    er = lax.dot_general(ar_ref[...], hp, (((1,), (1,)), ((), ())),
                         preferred_element_type=jnp.float32)
    zm = jnp.max(el) + jnp.max(er)
    m_sc = jnp.where(zm >= 0, zm, zm * 0.2)
    eler_ref[...] = jnp.concatenate(
        [el, er, jnp.full((1, el.shape[1]), m_sc, jnp.float32),
         jnp.zeros((5, el.shape[1]), jnp.float32)], axis=0)


def _tc_prep(s0, s1, d0, d1, badd, W, al, ar):
    return pl.pallas_call(
        _prep_body,
        out_shape=[
            jax.ShapeDtypeStruct((N_PAD, HID), jnp.float32),
            jax.ShapeDtypeStruct((8, N_PAD), jnp.float32),
        ],
    )(s0, s1, d0, d1, badd.reshape(1, -1), W,
      al.reshape(1, -1), ar.reshape(1, -1))


def _sc_gat_body(eidx_hbm, hp_hbm, eler_hbm, s_hbm, den_hbm,
                 el_t, er_t, m_t, idxb0, idxb1, wbuf, rows0, rows1,
                 out_s, den_s, semi0, semi1, semg0, semg1, *, ept):
    nch = ept // CH
    cid = lax.axis_index("c")
    sid = lax.axis_index("s")
    gwid = sid * NC + cid
    row0 = sid * NPT
    cbase = gwid * nch

    pltpu.sync_copy(eler_hbm.at[0], el_t)
    pltpu.sync_copy(eler_hbm.at[1], er_t)
    pltpu.sync_copy(eler_hbm.at[2, pl.ds(0, 16)], m_t)

    def _zr(i, _):
        rows0[i, :] = jnp.zeros((HID,), jnp.float32)
        return 0
    lax.fori_loop(0, CH, _zr, 0)

    def _zw(i, _):
        wbuf[pl.ds(i * 16, 16)] = jnp.zeros((16,), jnp.float32)
        return 0
    lax.fori_loop(0, CH // 16, _zw, 0)

    for k in range(NPT // CH):
        pltpu.sync_copy(rows0, out_s.at[pl.ds(row0 + k * CH, CH), :])
    for k in range(NPT // CH):
        pltpu.sync_copy(wbuf, den_s.at[pl.ds(row0 + k * CH, CH)])

    m_vec = m_t[...]

    plsc.subcore_barrier()

    def _phase(c, idxb, rows, semi_n, semg, idxb_n, nxt):
        pltpu.async_copy(hp_hbm.at[idxb.at[0]], rows, semg)
        pltpu.async_copy(eidx_hbm.at[nxt], idxb_n, semi_n)
        for j in range(CH // 16):
            s16 = idxb[0, pl.ds(j * 16, 16)]
            d16 = idxb[1, pl.ds(j * 16, 16)]
            zz = plsc.load_gather(el_t, [s16]) + plsc.load_gather(er_t, [d16])
            e = jnp.where(zz >= 0, zz, zz * 0.2)
            wbuf[pl.ds(j * 16, 16)] = jnp.exp(e - m_vec)
        pltpu.sync_copy(wbuf, den_s.at[idxb.at[1]], add=True)
        pltpu.make_async_copy(hp_hbm.at[idxb.at[0]], rows, semg).wait()
        for j in range(CH // 16):
            w16 = wbuf[pl.ds(j * 16, 16)]
            for k2 in range(16):
                jj = j * 16 + k2
                rows[jj, :] = rows[jj, :] * w16[k2]
        pltpu.sync_copy(rows, out_s.at[idxb.at[1]], add=True)

    pltpu.async_copy(eidx_hbm.at[cbase], idxb0, semi0)

    def _pair(i, _):
        cA = cbase + 2 * i
        pltpu.make_async_copy(eidx_hbm.at[cA], idxb0, semi0).wait()
        _phase(cA, idxb0, rows0, semi1, semg0, idxb1, cA + 1)
        pltpu.make_async_copy(eidx_hbm.at[cA + 1], idxb1, semi1).wait()
        _phase(cA + 1, idxb1, rows1, semi0, semg1, idxb0, cA + 2)
        return 0
    lax.fori_loop(0, nch // 2, _pair, 0)

    pltpu.make_async_copy(eidx_hbm.at[cbase + nch], idxb0, semi0).wait()

    plsc.subcore_barrier()

    pltpu.sync_copy(out_s.at[pl.ds(row0, NPT), :],
                    s_hbm.at[cid, pl.ds(row0, NPT), :])
    pltpu.sync_copy(den_s.at[pl.ds(row0, NPT)],
                    den_hbm.at[cid, pl.ds(row0, NPT)])


def _make_sc_gat(e_pad):
    ept = e_pad // (NC * NS)
    mesh = plsc.VectorSubcoreMesh(core_axis_name="c", subcore_axis_name="s",
                                  num_cores=NC)
    return pl.kernel(
        lambda *args: _sc_gat_body(*args, ept=ept),
        out_type=[
            jax.ShapeDtypeStruct((NC, N_PAD, HID), jnp.float32),
            jax.ShapeDtypeStruct((NC, N_PAD), jnp.float32),
        ],
        mesh=mesh,
        compiler_params=pltpu.CompilerParams(
            needs_layout_passes=False, use_tc_tiling_on_sc=False),
        scratch_types=[
            pltpu.VMEM((N_PAD,), jnp.float32),
            pltpu.VMEM((N_PAD,), jnp.float32),
            pltpu.VMEM((16,), jnp.float32),
            pltpu.VMEM((2, CH), jnp.int32),
            pltpu.VMEM((2, CH), jnp.int32),
            pltpu.VMEM((CH,), jnp.float32),
            pltpu.VMEM((CH, HID), jnp.float32),
            pltpu.VMEM((CH, HID), jnp.float32),
            pltpu.VMEM_SHARED((N_PAD, HID), jnp.float32),
            pltpu.VMEM_SHARED((N_PAD,), jnp.float32),
            pltpu.SemaphoreType.DMA,
            pltpu.SemaphoreType.DMA,
            pltpu.SemaphoreType.DMA,
            pltpu.SemaphoreType.DMA,
        ],
    )


def _lstm_body(sa0_ref, sa1_ref, da0_ref, da1_ref,
               sb0_ref, sb1_ref, db0_ref, db1_ref,
               h0_ref, b0_ref, b1_ref,
               wih0_ref, bb0_ref, wih1_ref, whh1_ref, bb1_ref,
               wp_ref, bp_ref, o_ref):
    t0 = (sa0_ref[...] + sa1_ref[...]) / (da0_ref[...] + da1_ref[...] + EPS)
    t1 = (sb0_ref[...] + sb1_ref[...]) / (db0_ref[...] + db1_ref[...] + EPS)
    xx = h0_ref[...]
    in0 = jnp.concatenate([t0 + b0_ref[...], xx], axis=-1)
    g = jnp.dot(in0, wih0_ref[...], preferred_element_type=jnp.float32) + bb0_ref[...]
    i, f, gg, o = jnp.split(g, 4, axis=-1)
    i = jax.nn.sigmoid(i)
    gg = jnp.tanh(gg)
    o = jax.nn.sigmoid(o)
    c1 = i * gg
    x1 = o * jnp.tanh(c1)
    in1 = jnp.concatenate([t1 + b1_ref[...], x1], axis=-1)
    g = (jnp.dot(in1, wih1_ref[...], preferred_element_type=jnp.float32)
         + jnp.dot(x1, whh1_ref[...], preferred_element_type=jnp.float32)
         + bb1_ref[...])
    i, f, gg, o = jnp.split(g, 4, axis=-1)
    i = jax.nn.sigmoid(i)
    f = jax.nn.sigmoid(f)
    gg = jnp.tanh(gg)
    o = jax.nn.sigmoid(o)
    c2 = f * c1 + i * gg
    x2 = o * jnp.tanh(c2)
    y = jnp.dot(x2, wp_ref[...], preferred_element_type=jnp.float32) + bp_ref[...]
    o_ref[...] = jax.nn.sigmoid(y)


def _tc_lstm(t0p, b0, t1p, b1, h0, W_ih0, b_ih0, b_hh0,
             W_ih1, W_hh1, b_ih1, b_hh1, W_pred, b_pred):
    wp128 = jnp.zeros((HID, 128), jnp.float32).at[:, 0].set(W_pred[:, 0])
    bp128 = jnp.zeros((1, 128), jnp.float32).at[0, 0].set(b_pred[0])
    sa0, sa1, da0, da1 = t0p
    sb0, sb1, db0, db1 = t1p
    blk = 2048
    node = lambda w: pl.BlockSpec((blk, w), lambda i: (i, 0))
    full = lambda a: pl.BlockSpec(a.shape, lambda i: (0,) * a.ndim)
    args = (sa0, sa1, da0, da1, sb0, sb1, db0, db1, h0,
            b0.reshape(1, -1), b1.reshape(1, -1),
            W_ih0.T, (b_ih0 + b_hh0).reshape(1, -1),
            W_ih1.T, W_hh1.T, (b_ih1 + b_hh1).reshape(1, -1),
            wp128, bp128)
    specs = [node(16), node(16), node(1), node(1),
             node(16), node(16), node(1), node(1), node(16)]
    specs += [full(a) for a in args[9:]]
    return pl.pallas_call(
        _lstm_body,
        grid=(N_PAD // blk,),
        in_specs=specs,
        out_specs=pl.BlockSpec((blk, 128), lambda i: (i, 0)),
        out_shape=jax.ShapeDtypeStruct((N_PAD, 128), jnp.float32),
    )(*args)


def kernel(x, edge_index, W_embed, b_embed, W_lin1, b_lin1,
           W_gat0, attn_l0, attn_r0, b_gat0,
           W_gat1, attn_l1, attn_r1, b_gat1,
           W_ih0, W_hh0, b_ih0, b_hh0,
           W_ih1, W_hh1, b_ih1, b_hh1,
           W_pred, b_pred):
    n, _ = x.shape
    e = edge_index.shape[1]
    grp = 2 * NC * NS * CH
    e_pad = ((e + grp - 1) // grp) * grp

    x_p = jnp.concatenate(
        [x, jnp.zeros((N_PAD - n, x.shape[1]), jnp.float32)], axis=0)
    pad_idx = jnp.full((e_pad - e,), N_PAD - 1, jnp.int32)
    src_p = jnp.concatenate([edge_index[0], pad_idx])
    dst_p = jnp.concatenate([edge_index[1], pad_idx])
    eidx = jnp.stack([src_p.reshape(-1, CH), dst_p.reshape(-1, CH)], axis=1)
    eidx = jnp.concatenate(
        [eidx, jnp.full((1, 2, CH), N_PAD - 1, jnp.int32)], axis=0)

    sc_gat = _make_sc_gat(e_pad)

    h0 = _tc_embed(x_p, W_embed, b_embed, W_lin1, b_lin1)

    zb = jnp.zeros((HID,), jnp.float32)
    zs = jnp.zeros((N_PAD, HID), jnp.float32)
    zd = jnp.zeros((N_PAD, 1), jnp.float32)
    od = jnp.ones((N_PAD, 1), jnp.float32)
    h0_parts = (h0, zs, od, zd)

    def gat(parts, badd, W, al, ar):
        hp, eler = _tc_prep(*parts, badd, W, al, ar)
        s, den = sc_gat(eidx, hp, eler)
        return (s[0], s[1], den[0].reshape(N_PAD, 1), den[1].reshape(N_PAD, 1))

    t0p = gat(h0_parts, zb, W_gat0, attn_l0, attn_r0)
    t0p = gat(t0p, b_gat0, W_gat0, attn_l0, attn_r0)
    t1p = gat(h0_parts, zb, W_gat1, attn_l1, attn_r1)
    t1p = gat(t1p, b_gat1, W_gat1, attn_l1, attn_r1)

    y = _tc_lstm(t0p, b_gat0, t1p, b_gat1, h0,
                 W_ih0, b_ih0, b_hh0, W_ih1, W_hh1, b_ih1, b_hh1,
                 W_pred, b_pred)
    return y[:n, :1]


# speedup vs baseline: 60.1265x; 1.0077x over previous
import jax
import jax.numpy as jnp
from jax import lax
from jax.experimental import pallas as pl
from jax.experimental.pallas import tpu as pltpu
from jax.experimental.pallas import tpu_sc as plsc

N_PAD = 10240
HID = 16
NC = 2
NS = 16
NPT = N_PAD // NS
CH = 128
EPS = 1e-9


def _embed_body(x_ref, we_ref, be_ref, wl_ref, bl_ref, o_ref):
    h = jnp.dot(x_ref[...], we_ref[...], preferred_element_type=jnp.float32)
    h = h + be_ref[...]
    o_ref[...] = (
        jnp.dot(h, wl_ref[...], preferred_element_type=jnp.float32) + bl_ref[...]
    )


def _tc_embed(x_p, W_embed, b_embed, W_lin1, b_lin1):
    return pl.pallas_call(
        _embed_body,
        out_shape=jax.ShapeDtypeStruct((N_PAD, HID), jnp.float32),
    )(x_p, W_embed, b_embed.reshape(1, -1), W_lin1, b_lin1.reshape(1, -1))


def _prep_body(s0_ref, s1_ref, d0_ref, d1_ref, b_ref, W_ref, al_ref, ar_ref,
               hp_ref, eler_ref):
    h = (s0_ref[...] + s1_ref[...]) / (d0_ref[...] + d1_ref[...] + EPS)
    h = h + b_ref[...]
    hp = jnp.dot(h, W_ref[...], preferred_element_type=jnp.float32)
    hp_ref[...] = hp
    el = lax.dot_general(al_ref[...], hp, (((1,), (1,)), ((), ())),
                         preferred_element_type=jnp.float32)
    er = lax.dot_general(ar_ref[...], hp, (((1,), (1,)), ((), ())),
                         preferred_element_type=jnp.float32)
    zm = jnp.max(el) + jnp.max(er)
    m_sc = jnp.where(zm >= 0, zm, zm * 0.2)
    eler_ref[...] = jnp.concatenate(
        [el, er, jnp.full((1, el.shape[1]), m_sc, jnp.float32),
         jnp.zeros((5, el.shape[1]), jnp.float32)], axis=0)


def _tc_prep(s0, s1, d0, d1, badd, W, al, ar):
    return pl.pallas_call(
        _prep_body,
        out_shape=[
            jax.ShapeDtypeStruct((N_PAD, HID), jnp.float32),
            jax.ShapeDtypeStruct((8, N_PAD), jnp.float32),
        ],
    )(s0, s1, d0, d1, badd.reshape(1, -1), W,
      al.reshape(1, -1), ar.reshape(1, -1))


def _sc_gat_body(eidx_hbm, hp_hbm, eler_hbm, s_hbm, den_hbm,
                 el_t, er_t, m_t, idxb, wbuf, rows,
                 out_s, den_s, semi, semg, semd, semo, *, ept):
    nch = ept // CH
    cid = lax.axis_index("c")
    sid = lax.axis_index("s")
    gwid = sid * NC + cid
    row0 = sid * NPT
    cbase = gwid * nch

    pltpu.sync_copy(eler_hbm.at[0], el_t)
    pltpu.sync_copy(eler_hbm.at[1], er_t)
    pltpu.sync_copy(eler_hbm.at[2, pl.ds(0, 16)], m_t)

    def _zr(i, _):
        rows[0][i, :] = jnp.zeros((HID,), jnp.float32)
        return 0
    lax.fori_loop(0, CH, _zr, 0)

    def _zw(i, _):
        wbuf[0][pl.ds(i * 16, 16)] = jnp.zeros((16,), jnp.float32)
        return 0
    lax.fori_loop(0, CH // 16, _zw, 0)

    for k in range(NPT // CH):
        pltpu.sync_copy(rows[0], out_s.at[pl.ds(row0 + k * CH, CH), :])
    for k in range(NPT // CH):
        pltpu.sync_copy(wbuf[0], den_s.at[pl.ds(row0 + k * CH, CH)])

    m_vec = m_t[...]

    plsc.subcore_barrier()

    def _drain_scatters(k):
        pltpu.make_async_copy(wbuf[k], den_s.at[idxb[k].at[1]], semd[k]).wait()
        pltpu.make_async_copy(rows[k], out_s.at[idxb[k].at[1]], semo[k]).wait()

    def _phase(c, k, has_prev):
        k1 = (k + 1) % 3
        pltpu.make_async_copy(eidx_hbm.at[c], idxb[k], semi[k]).wait()
        pltpu.async_copy(hp_hbm.at[idxb[k].at[0]], rows[k], semg[k])
        if has_prev:
            _drain_scatters(k1)
        pltpu.async_copy(eidx_hbm.at[c + 1], idxb[k1], semi[k1])
        for j in range(CH // 16):
            s16 = idxb[k][0, pl.ds(j * 16, 16)]
            d16 = idxb[k][1, pl.ds(j * 16, 16)]
            zz = plsc.load_gather(el_t, [s16]) + plsc.load_gather(er_t, [d16])
            e = jnp.where(zz >= 0, zz, zz * 0.2)
            wbuf[k][pl.ds(j * 16, 16)] = jnp.exp(e - m_vec)
        pltpu.async_copy(wbuf[k], den_s.at[idxb[k].at[1]], semd[k], add=True)
        pltpu.make_async_copy(hp_hbm.at[idxb[k].at[0]], rows[k], semg[k]).wait()
        for j in range(CH // 16):
            w16 = wbuf[k][pl.ds(j * 16, 16)]
            for k2 in range(16):
                jj = j * 16 + k2
                rows[k][jj, :] = rows[k][jj, :] * w16[k2]
        pltpu.async_copy(rows[k], out_s.at[idxb[k].at[1]], semo[k], add=True)

    pltpu.async_copy(eidx_hbm.at[cbase], idxb[0], semi[0])
    _phase(cbase + 0, 0, False)
    _phase(cbase + 1, 1, False)
    _phase(cbase + 2, 2, True)

    def _trip(i, _):
        c = cbase + 3 * i
        _phase(c + 0, 0, True)
        _phase(c + 1, 1, True)
        _phase(c + 2, 2, True)
        return 0
    lax.fori_loop(1, nch // 3, _trip, 0)

    pltpu.make_async_copy(eidx_hbm.at[cbase + nch], idxb[0], semi[0]).wait()
    _drain_scatters((nch - 2) % 3)
    _drain_scatters((nch - 1) % 3)

    plsc.subcore_barrier()

    pltpu.sync_copy(out_s.at[pl.ds(row0, NPT), :],
                    s_hbm.at[cid, pl.ds(row0, NPT), :])
    pltpu.sync_copy(den_s.at[pl.ds(row0, NPT)],
                    den_hbm.at[cid, pl.ds(row0, NPT)])


def _make_sc_gat(e_pad):
    ept = e_pad // (NC * NS)
    mesh = plsc.VectorSubcoreMesh(core_axis_name="c", subcore_axis_name="s",
                                  num_cores=NC)
    return pl.kernel(
        lambda *args: _sc_gat_body(*args, ept=ept),
        out_type=[
            jax.ShapeDtypeStruct((NC, N_PAD, HID), jnp.float32),
            jax.ShapeDtypeStruct((NC, N_PAD), jnp.float32),
        ],
        mesh=mesh,
        compiler_params=pltpu.CompilerParams(
            needs_layout_passes=False, use_tc_tiling_on_sc=False),
        scratch_types=[
            pltpu.VMEM((N_PAD,), jnp.float32),
            pltpu.VMEM((N_PAD,), jnp.float32),
            pltpu.VMEM((16,), jnp.float32),
            [pltpu.VMEM((2, CH), jnp.int32) for _ in range(3)],
            [pltpu.VMEM((CH,), jnp.float32) for _ in range(3)],
            [pltpu.VMEM((CH, HID), jnp.float32) for _ in range(3)],
            pltpu.VMEM_SHARED((N_PAD, HID), jnp.float32),
            pltpu.VMEM_SHARED((N_PAD,), jnp.float32),
            [pltpu.SemaphoreType.DMA for _ in range(3)],
            [pltpu.SemaphoreType.DMA for _ in range(3)],
            [pltpu.SemaphoreType.DMA for _ in range(3)],
            [pltpu.SemaphoreType.DMA for _ in range(3)],
        ],
    )


def _lstm_body(sa0_ref, sa1_ref, da0_ref, da1_ref,
               sb0_ref, sb1_ref, db0_ref, db1_ref,
               h0_ref, b0_ref, b1_ref,
               wih0_ref, bb0_ref, wih1_ref, whh1_ref, bb1_ref,
               wp_ref, bp_ref, o_ref):
    t0 = (sa0_ref[...] + sa1_ref[...]) / (da0_ref[...] + da1_ref[...] + EPS)
    t1 = (sb0_ref[...] + sb1_ref[...]) / (db0_ref[...] + db1_ref[...] + EPS)
    xx = h0_ref[...]
    in0 = jnp.concatenate([t0 + b0_ref[...], xx], axis=-1)
    g = jnp.dot(in0, wih0_ref[...], preferred_element_type=jnp.float32) + bb0_ref[...]
    i, f, gg, o = jnp.split(g, 4, axis=-1)
    i = jax.nn.sigmoid(i)
    gg = jnp.tanh(gg)
    o = jax.nn.sigmoid(o)
    c1 = i * gg
    x1 = o * jnp.tanh(c1)
    in1 = jnp.concatenate([t1 + b1_ref[...], x1], axis=-1)
    g = (jnp.dot(in1, wih1_ref[...], preferred_element_type=jnp.float32)
         + jnp.dot(x1, whh1_ref[...], preferred_element_type=jnp.float32)
         + bb1_ref[...])
    i, f, gg, o = jnp.split(g, 4, axis=-1)
    i = jax.nn.sigmoid(i)
    f = jax.nn.sigmoid(f)
    gg = jnp.tanh(gg)
    o = jax.nn.sigmoid(o)
    c2 = f * c1 + i * gg
    x2 = o * jnp.tanh(c2)
    y = jnp.dot(x2, wp_ref[...], preferred_element_type=jnp.float32) + bp_ref[...]
    o_ref[...] = jax.nn.sigmoid(y)


def _tc_lstm(t0p, b0, t1p, b1, h0, W_ih0, b_ih0, b_hh0,
             W_ih1, W_hh1, b_ih1, b_hh1, W_pred, b_pred):
    wp128 = jnp.zeros((HID, 128), jnp.float32).at[:, 0].set(W_pred[:, 0])
    bp128 = jnp.zeros((1, 128), jnp.float32).at[0, 0].set(b_pred[0])
    sa0, sa1, da0, da1 = t0p
    sb0, sb1, db0, db1 = t1p
    blk = 2048
    node = lambda w: pl.BlockSpec((blk, w), lambda i: (i, 0))
    full = lambda a: pl.BlockSpec(a.shape, lambda i: (0,) * a.ndim)
    args = (sa0, sa1, da0, da1, sb0, sb1, db0, db1, h0,
            b0.reshape(1, -1), b1.reshape(1, -1),
            W_ih0.T, (b_ih0 + b_hh0).reshape(1, -1),
            W_ih1.T, W_hh1.T, (b_ih1 + b_hh1).reshape(1, -1),
            wp128, bp128)
    specs = [node(16), node(16), node(1), node(1),
             node(16), node(16), node(1), node(1), node(16)]
    specs += [full(a) for a in args[9:]]
    return pl.pallas_call(
        _lstm_body,
        grid=(N_PAD // blk,),
        in_specs=specs,
        out_specs=pl.BlockSpec((blk, 128), lambda i: (i, 0)),
        out_shape=jax.ShapeDtypeStruct((N_PAD, 128), jnp.float32),
    )(*args)


def kernel(x, edge_index, W_embed, b_embed, W_lin1, b_lin1,
           W_gat0, attn_l0, attn_r0, b_gat0,
           W_gat1, attn_l1, attn_r1, b_gat1,
           W_ih0, W_hh0, b_ih0, b_hh0,
           W_ih1, W_hh1, b_ih1, b_hh1,
           W_pred, b_pred):
    n, _ = x.shape
    e = edge_index.shape[1]
    grp = 3 * NC * NS * CH
    e_pad = ((e + grp - 1) // grp) * grp

    x_p = jnp.concatenate(
        [x, jnp.zeros((N_PAD - n, x.shape[1]), jnp.float32)], axis=0)
    pad_idx = jnp.full((e_pad - e,), N_PAD - 1, jnp.int32)
    src_p = jnp.concatenate([edge_index[0], pad_idx])
    dst_p = jnp.concatenate([edge_index[1], pad_idx])
    eidx = jnp.stack([src_p.reshape(-1, CH), dst_p.reshape(-1, CH)], axis=1)
    eidx = jnp.concatenate(
        [eidx, jnp.full((1, 2, CH), N_PAD - 1, jnp.int32)], axis=0)

    sc_gat = _make_sc_gat(e_pad)

    h0 = _tc_embed(x_p, W_embed, b_embed, W_lin1, b_lin1)

    zb = jnp.zeros((HID,), jnp.float32)
    zs = jnp.zeros((N_PAD, HID), jnp.float32)
    zd = jnp.zeros((N_PAD, 1), jnp.float32)
    od = jnp.ones((N_PAD, 1), jnp.float32)
    h0_parts = (h0, zs, od, zd)

    def gat(parts, badd, W, al, ar):
        hp, eler = _tc_prep(*parts, badd, W, al, ar)
        s, den = sc_gat(eidx, hp, eler)
        return (s[0], s[1], den[0].reshape(N_PAD, 1), den[1].reshape(N_PAD, 1))

    hpA, elerA = _tc_prep(*h0_parts, zb, W_gat0, attn_l0, attn_r0)
    hpB, elerB = _tc_prep(*h0_parts, zb, W_gat1, attn_l1, attn_r1)
    sA, dA = sc_gat(eidx, hpA, elerA)
    sB, dB = sc_gat(eidx, hpB, elerB)
    t0p = (sA[0], sA[1], dA[0].reshape(N_PAD, 1), dA[1].reshape(N_PAD, 1))
    t1p = (sB[0], sB[1], dB[0].reshape(N_PAD, 1), dB[1].reshape(N_PAD, 1))
    t0p = gat(t0p, b_gat0, W_gat0, attn_l0, attn_r0)
    t1p = gat(t1p, b_gat1, W_gat1, attn_l1, attn_r1)

    y = _tc_lstm(t0p, b_gat0, t1p, b_gat1, h0,
                 W_ih0, b_ih0, b_hh0, W_ih1, W_hh1, b_ih1, b_hh1,
                 W_pred, b_pred)
    return y[:n, :1]


# speedup vs baseline: 89.8929x; 1.4951x over previous
import jax
import jax.numpy as jnp
from jax import lax
from jax.experimental import pallas as pl
from jax.experimental.pallas import tpu as pltpu
from jax.experimental.pallas import tpu_sc as plsc

N_PAD = 10240
HID = 16
NC = 2
NS = 16
NPT = N_PAD // NS
CH = 128
EPS = 1e-9


def _embed_body(x_ref, we_ref, be_ref, wl_ref, bl_ref, o_ref):
    h = jnp.dot(x_ref[...], we_ref[...], preferred_element_type=jnp.float32)
    h = h + be_ref[...]
    o_ref[...] = (
        jnp.dot(h, wl_ref[...], preferred_element_type=jnp.float32) + bl_ref[...]
    )


def _tc_embed(x_p, W_embed, b_embed, W_lin1, b_lin1):
    return pl.pallas_call(
        _embed_body,
        out_shape=jax.ShapeDtypeStruct((N_PAD, HID), jnp.float32),
    )(x_p, W_embed, b_embed.reshape(1, -1), W_lin1, b_lin1.reshape(1, -1))


def _prep_body(s0_ref, s1_ref, d0_ref, d1_ref, b_ref, W_ref, al_ref, ar_ref,
               hp_ref, eler_ref):
    h = (s0_ref[...] + s1_ref[...]) / (d0_ref[...] + d1_ref[...] + EPS)
    h = h + b_ref[...]
    hp = jnp.dot(h, W_ref[...], preferred_element_type=jnp.float32)
    hp_ref[...] = hp
    el = lax.dot_general(al_ref[...], hp, (((1,), (1,)), ((), ())),
                         preferred_element_type=jnp.float32)
    er = lax.dot_general(ar_ref[...], hp, (((1,), (1,)), ((), ())),
                         preferred_element_type=jnp.float32)
    zm = jnp.max(el) + jnp.max(er)
    m_sc = jnp.where(zm >= 0, zm, zm * 0.2)
    eler_ref[...] = jnp.concatenate(
        [el, er, jnp.full((1, el.shape[1]), m_sc, jnp.float32),
         jnp.zeros((5, el.shape[1]), jnp.float32)], axis=0)


def _tc_prep(s0, s1, d0, d1, badd, W, al, ar):
    return pl.pallas_call(
        _prep_body,
        out_shape=[
            jax.ShapeDtypeStruct((N_PAD, HID), jnp.float32),
            jax.ShapeDtypeStruct((8, N_PAD), jnp.float32),
        ],
    )(s0, s1, d0, d1, badd.reshape(1, -1), W,
      al.reshape(1, -1), ar.reshape(1, -1))


def _sc_gat_body(eidx_hbm, hp_hbm, eler_hbm, s_hbm, den_hbm,
                 el_t, er_t, m_t, idxb, wbuf, rows,
                 out_s, den_s, semi, semg, semd, semo, *, ept):
    nch = ept // CH
    cid = lax.axis_index("c")
    sid = lax.axis_index("s")
    gwid = sid * NC + cid
    row0 = sid * NPT
    cbase = gwid * nch

    pltpu.sync_copy(eler_hbm.at[0], el_t)
    pltpu.sync_copy(eler_hbm.at[1], er_t)
    pltpu.sync_copy(eler_hbm.at[2, pl.ds(0, 16)], m_t)

    def _zr(i, _):
        rows[0][i, :] = jnp.zeros((HID,), jnp.float32)
        return 0
    lax.fori_loop(0, CH, _zr, 0)

    def _zw(i, _):
        wbuf[0][pl.ds(i * 16, 16)] = jnp.zeros((16,), jnp.float32)
        return 0
    lax.fori_loop(0, CH // 16, _zw, 0)

    for k in range(NPT // CH):
        pltpu.sync_copy(rows[0], out_s.at[pl.ds(row0 + k * CH, CH), :])
    for k in range(NPT // CH):
        pltpu.sync_copy(wbuf[0], den_s.at[pl.ds(row0 + k * CH, CH)])

    m_vec = m_t[...]

    plsc.subcore_barrier()

    def _drain_scatters(k):
        pltpu.make_async_copy(wbuf[k], den_s.at[idxb[k].at[1]], semd[k]).wait()
        pltpu.make_async_copy(rows[k], out_s.at[idxb[k].at[1]], semo[k]).wait()

    def _phase(c, k, has_prev):
        k1 = (k + 1) % 3
        k2 = (k + 2) % 3
        pltpu.make_async_copy(eidx_hbm.at[c + 1], idxb[k1], semi[k1]).wait()
        pltpu.async_copy(hp_hbm.at[idxb[k1].at[0]], rows[k1], semg[k1])
        if has_prev:
            _drain_scatters(k2)
        pltpu.async_copy(eidx_hbm.at[c + 2], idxb[k2], semi[k2])
        for j in range(CH // 16):
            s16 = idxb[k][0, pl.ds(j * 16, 16)]
            d16 = idxb[k][1, pl.ds(j * 16, 16)]
            zz = plsc.load_gather(el_t, [s16]) + plsc.load_gather(er_t, [d16])
            e = jnp.where(zz >= 0, zz, zz * 0.2)
            wbuf[k][pl.ds(j * 16, 16)] = jnp.exp(e - m_vec)
        pltpu.async_copy(wbuf[k], den_s.at[idxb[k].at[1]], semd[k], add=True)
        pltpu.make_async_copy(hp_hbm.at[idxb[k].at[0]], rows[k], semg[k]).wait()
        for j in range(CH // 16):
            w16 = wbuf[k][pl.ds(j * 16, 16)]
            for k2_ in range(16):
                jj = j * 16 + k2_
                rows[k][jj, :] = rows[k][jj, :] * w16[k2_]
        pltpu.async_copy(rows[k], out_s.at[idxb[k].at[1]], semo[k], add=True)

    pltpu.async_copy(eidx_hbm.at[cbase], idxb[0], semi[0])
    pltpu.make_async_copy(eidx_hbm.at[cbase], idxb[0], semi[0]).wait()
    pltpu.async_copy(hp_hbm.at[idxb[0].at[0]], rows[0], semg[0])
    pltpu.async_copy(eidx_hbm.at[cbase + 1], idxb[1], semi[1])
    _phase(cbase + 0, 0, False)
    _phase(cbase + 1, 1, True)
    _phase(cbase + 2, 2, True)
    _phase(cbase + 3, 0, True)

    def _trip(i, _):
        c = cbase + 4 + 3 * i
        _phase(c + 0, 1, True)
        _phase(c + 1, 2, True)
        _phase(c + 2, 0, True)
        return 0
    lax.fori_loop(0, (nch - 4) // 3, _trip, 0)

    kg = nch % 3
    ki = (nch + 1) % 3
    kl = (nch - 1) % 3
    pltpu.make_async_copy(hp_hbm.at[idxb[kg].at[0]], rows[kg], semg[kg]).wait()
    pltpu.make_async_copy(eidx_hbm.at[cbase + nch + 1], idxb[ki], semi[ki]).wait()
    _drain_scatters(kl)

    plsc.subcore_barrier()

    pltpu.sync_copy(out_s.at[pl.ds(row0, NPT), :],
                    s_hbm.at[cid, pl.ds(row0, NPT), :])
    pltpu.sync_copy(den_s.at[pl.ds(row0, NPT)],
                    den_hbm.at[cid, pl.ds(row0, NPT)])


def _make_sc_gat(e_pad):
    ept = e_pad // (NC * NS)
    mesh = plsc.VectorSubcoreMesh(core_axis_name="c", subcore_axis_name="s",
                                  num_cores=NC)
    return pl.kernel(
        lambda *args: _sc_gat_body(*args, ept=ept),
        out_type=[
            jax.ShapeDtypeStruct((NC, N_PAD, HID), jnp.float32),
            jax.ShapeDtypeStruct((NC, N_PAD), jnp.float32),
        ],
        mesh=mesh,
        compiler_params=pltpu.CompilerParams(
            needs_layout_passes=False, use_tc_tiling_on_sc=False),
        scratch_types=[
            pltpu.VMEM((N_PAD,), jnp.float32),
            pltpu.VMEM((N_PAD,), jnp.float32),
            pltpu.VMEM((16,), jnp.float32),
            [pltpu.VMEM((2, CH), jnp.int32) for _ in range(3)],
            [pltpu.VMEM((CH,), jnp.float32) for _ in range(3)],
            [pltpu.VMEM((CH, HID), jnp.float32) for _ in range(3)],
            pltpu.VMEM_SHARED((N_PAD, HID), jnp.float32),
            pltpu.VMEM_SHARED((N_PAD,), jnp.float32),
            [pltpu.SemaphoreType.DMA for _ in range(3)],
            [pltpu.SemaphoreType.DMA for _ in range(3)],
            [pltpu.SemaphoreType.DMA for _ in range(3)],
            [pltpu.SemaphoreType.DMA for _ in range(3)],
        ],
    )


def _lstm_body(sa0_ref, sa1_ref, da0_ref, da1_ref,
               sb0_ref, sb1_ref, db0_ref, db1_ref,
               h0_ref, b0_ref, b1_ref,
               wih0_ref, bb0_ref, wih1_ref, whh1_ref, bb1_ref,
               wp_ref, bp_ref, o_ref):
    t0 = (sa0_ref[...] + sa1_ref[...]) / (da0_ref[...] + da1_ref[...] + EPS)
    t1 = (sb0_ref[...] + sb1_ref[...]) / (db0_ref[...] + db1_ref[...] + EPS)
    xx = h0_ref[...]
    in0 = jnp.concatenate([t0 + b0_ref[...], xx], axis=-1)
    g = jnp.dot(in0, wih0_ref[...], preferred_element_type=jnp.float32) + bb0_ref[...]
    i, f, gg, o = jnp.split(g, 4, axis=-1)
    i = jax.nn.sigmoid(i)
    gg = jnp.tanh(gg)
    o = jax.nn.sigmoid(o)
    c1 = i * gg
    x1 = o * jnp.tanh(c1)
    in1 = jnp.concatenate([t1 + b1_ref[...], x1], axis=-1)
    g = (jnp.dot(in1, wih1_ref[...], preferred_element_type=jnp.float32)
         + jnp.dot(x1, whh1_ref[...], preferred_element_type=jnp.float32)
         + bb1_ref[...])
    i, f, gg, o = jnp.split(g, 4, axis=-1)
    i = jax.nn.sigmoid(i)
    f = jax.nn.sigmoid(f)
    gg = jnp.tanh(gg)
    o = jax.nn.sigmoid(o)
    c2 = f * c1 + i * gg
    x2 = o * jnp.tanh(c2)
    y = jnp.dot(x2, wp_ref[...], preferred_element_type=jnp.float32) + bp_ref[...]
    o_ref[...] = jax.nn.sigmoid(y)


def _tc_lstm(t0p, b0, t1p, b1, h0, W_ih0, b_ih0, b_hh0,
             W_ih1, W_hh1, b_ih1, b_hh1, W_pred, b_pred):
    wp128 = jnp.zeros((HID, 128), jnp.float32).at[:, 0].set(W_pred[:, 0])
    bp128 = jnp.zeros((1, 128), jnp.float32).at[0, 0].set(b_pred[0])
    sa0, sa1, da0, da1 = t0p
    sb0, sb1, db0, db1 = t1p
    blk = 2048
    node = lambda w: pl.BlockSpec((blk, w), lambda i: (i, 0))
    full = lambda a: pl.BlockSpec(a.shape, lambda i: (0,) * a.ndim)
    args = (sa0, sa1, da0, da1, sb0, sb1, db0, db1, h0,
            b0.reshape(1, -1), b1.reshape(1, -1),
            W_ih0.T, (b_ih0 + b_hh0).reshape(1, -1),
            W_ih1.T, W_hh1.T, (b_ih1 + b_hh1).reshape(1, -1),
            wp128, bp128)
    specs = [node(16), node(16), node(1), node(1),
             node(16), node(16), node(1), node(1), node(16)]
    specs += [full(a) for a in args[9:]]
    return pl.pallas_call(
        _lstm_body,
        grid=(N_PAD // blk,),
        in_specs=specs,
        out_specs=pl.BlockSpec((blk, 128), lambda i: (i, 0)),
        out_shape=jax.ShapeDtypeStruct((N_PAD, 128), jnp.float32),
    )(*args)


def kernel(x, edge_index, W_embed, b_embed, W_lin1, b_lin1,
           W_gat0, attn_l0, attn_r0, b_gat0,
           W_gat1, attn_l1, attn_r1, b_gat1,
           W_ih0, W_hh0, b_ih0, b_hh0,
           W_ih1, W_hh1, b_ih1, b_hh1,
           W_pred, b_pred):
    n, _ = x.shape
    e = edge_index.shape[1]
    grp = NC * NS * CH
    nch = (e + grp - 1) // grp
    while nch % 3 != 1 or nch < 7:
        nch += 1
    e_pad = nch * grp

    x_p = jnp.concatenate(
        [x, jnp.zeros((N_PAD - n, x.shape[1]), jnp.float32)], axis=0)
    pad_idx = jnp.full((e_pad - e,), N_PAD - 1, jnp.int32)
    src_p = jnp.concatenate([edge_index[0], pad_idx])
    dst_p = jnp.concatenate([edge_index[1], pad_idx])
    eidx = jnp.stack([src_p.reshape(-1, CH), dst_p.reshape(-1, CH)], axis=1)
    eidx = jnp.concatenate(
        [eidx, jnp.full((2, 2, CH), N_PAD - 1, jnp.int32)], axis=0)

    sc_gat = _make_sc_gat(e_pad)

    h0 = _tc_embed(x_p, W_embed, b_embed, W_lin1, b_lin1)

    zb = jnp.zeros((HID,), jnp.float32)
    zs = jnp.zeros((N_PAD, HID), jnp.float32)
    zd = jnp.zeros((N_PAD, 1), jnp.float32)
    od = jnp.ones((N_PAD, 1), jnp.float32)
    h0_parts = (h0, zs, od, zd)

    def gat(parts, badd, W, al, ar):
        hp, eler = _tc_prep(*parts, badd, W, al, ar)
        s, den = sc_gat(eidx, hp, eler)
        return (s[0], s[1], den[0].reshape(N_PAD, 1), den[1].reshape(N_PAD, 1))

    hpA, elerA = _tc_prep(*h0_parts, zb, W_gat0, attn_l0, attn_r0)
    hpB, elerB = _tc_prep(*h0_parts, zb, W_gat1, attn_l1, attn_r1)
    sA, dA = sc_gat(eidx, hpA, elerA)
    sB, dB = sc_gat(eidx, hpB, elerB)
    t0p = (sA[0], sA[1], dA[0].reshape(N_PAD, 1), dA[1].reshape(N_PAD, 1))
    t1p = (sB[0], sB[1], dB[0].reshape(N_PAD, 1), dB[1].reshape(N_PAD, 1))
    t0p = gat(t0p, b_gat0, W_gat0, attn_l0, attn_r0)
    t1p = gat(t1p, b_gat1, W_gat1, attn_l1, attn_r1)

    y = _tc_lstm(t0p, b_gat0, t1p, b_gat1, h0,
                 W_ih0, b_ih0, b_hh0, W_ih1, W_hh1, b_ih1, b_hh1,
                 W_pred, b_pred)
    return y[:n, :1]


# speedup vs baseline: 114.2988x; 1.2715x over previous
import jax
import jax.numpy as jnp
from jax import lax
from jax.experimental import pallas as pl
from jax.experimental.pallas import tpu as pltpu
from jax.experimental.pallas import tpu_sc as plsc

N_PAD = 10240
HID = 16
NC = 2
NS = 16
NPT = N_PAD // NS
CH = 128
EPS = 1e-9


def _proj_logits(h, W_ref, al_ref, ar_ref):
    hp = jnp.dot(h, W_ref[...], preferred_element_type=jnp.float32)
    el = lax.dot_general(al_ref[...], hp, (((1,), (1,)), ((), ())),
                         preferred_element_type=jnp.float32)
    er = lax.dot_general(ar_ref[...], hp, (((1,), (1,)), ((), ())),
                         preferred_element_type=jnp.float32)
    zm = jnp.max(el) + jnp.max(er)
    m_sc = jnp.where(zm >= 0, zm, zm * 0.2)
    eler = jnp.concatenate(
        [el, er, jnp.full((1, el.shape[1]), m_sc, jnp.float32),
         jnp.zeros((5, el.shape[1]), jnp.float32)], axis=0)
    return hp, eler


def _embed_body(x_ref, we_ref, be_ref, wl_ref, bl_ref,
                wA_ref, alA_ref, arA_ref, wB_ref, alB_ref, arB_ref,
                h0_ref, hp_ref, eler_ref):
    h = jnp.dot(x_ref[...], we_ref[...], preferred_element_type=jnp.float32)
    h = h + be_ref[...]
    h0 = jnp.dot(h, wl_ref[...], preferred_element_type=jnp.float32) + bl_ref[...]
    h0_ref[...] = h0
    hpA, elerA = _proj_logits(h0, wA_ref, alA_ref, arA_ref)
    hpB, elerB = _proj_logits(h0, wB_ref, alB_ref, arB_ref)
    hp_ref[0:N_PAD, :] = hpA
    hp_ref[N_PAD:, :] = hpB
    eler_ref[0] = elerA
    eler_ref[1] = elerB


def _tc_embed(x_p, W_embed, b_embed, W_lin1, b_lin1,
              W_gat0, attn_l0, attn_r0, W_gat1, attn_l1, attn_r1):
    return pl.pallas_call(
        _embed_body,
        out_shape=[
            jax.ShapeDtypeStruct((N_PAD, HID), jnp.float32),
            jax.ShapeDtypeStruct((2 * N_PAD, HID), jnp.float32),
            jax.ShapeDtypeStruct((2, 8, N_PAD), jnp.float32),
        ],
    )(x_p, W_embed, b_embed.reshape(1, -1), W_lin1, b_lin1.reshape(1, -1),
      W_gat0, attn_l0.reshape(1, -1), attn_r0.reshape(1, -1),
      W_gat1, attn_l1.reshape(1, -1), attn_r1.reshape(1, -1))


def _prep2_body(s_ref, dA_ref, dB_ref, bA_ref, bB_ref,
                wA_ref, alA_ref, arA_ref, wB_ref, alB_ref, arB_ref,
                hp_ref, eler_ref):
    hA = s_ref[0] / (dA_ref[...] + EPS) + bA_ref[...]
    hB = s_ref[1] / (dB_ref[...] + EPS) + bB_ref[...]
    hpA, elerA = _proj_logits(hA, wA_ref, alA_ref, arA_ref)
    hpB, elerB = _proj_logits(hB, wB_ref, alB_ref, arB_ref)
    hp_ref[0:N_PAD, :] = hpA
    hp_ref[N_PAD:, :] = hpB
    eler_ref[0] = elerA
    eler_ref[1] = elerB


def _tc_prep2(s, dA, dB, bA, bB,
              W_gat0, attn_l0, attn_r0, W_gat1, attn_l1, attn_r1):
    return pl.pallas_call(
        _prep2_body,
        out_shape=[
            jax.ShapeDtypeStruct((2 * N_PAD, HID), jnp.float32),
            jax.ShapeDtypeStruct((2, 8, N_PAD), jnp.float32),
        ],
    )(s, dA, dB, bA.reshape(1, -1), bB.reshape(1, -1),
      W_gat0, attn_l0.reshape(1, -1), attn_r0.reshape(1, -1),
      W_gat1, attn_l1.reshape(1, -1), attn_r1.reshape(1, -1))


def _sc_gat_body(eidx_hbm, hp_hbm, eler_hbm, s_hbm, den_hbm,
                 el_t, er_t, m_t, idxb, wbuf, rows,
                 out_s, den_s, semi, semg, semd, semo, *, nch):
    cid = lax.axis_index("c")
    sid = lax.axis_index("s")
    row0 = sid * NPT
    cbase = sid * nch

    pltpu.sync_copy(eler_hbm.at[cid, 0], el_t.at[pl.ds(cid * N_PAD, N_PAD)])
    pltpu.sync_copy(eler_hbm.at[cid, 1], er_t)
    pltpu.sync_copy(eler_hbm.at[cid, 2, pl.ds(0, 16)], m_t)

    def _zr(i, _):
        rows[0][i, :] = jnp.zeros((HID,), jnp.float32)
        return 0
    lax.fori_loop(0, CH, _zr, 0)

    def _zw(i, _):
        wbuf[0][pl.ds(i * 16, 16)] = jnp.zeros((16,), jnp.float32)
        return 0
    lax.fori_loop(0, CH // 16, _zw, 0)

    for k in range(NPT // CH):
        pltpu.sync_copy(rows[0], out_s.at[pl.ds(row0 + k * CH, CH), :])
    for k in range(NPT // CH):
        pltpu.sync_copy(wbuf[0], den_s.at[pl.ds(row0 + k * CH, CH)])

    m_vec = m_t[...]

    plsc.subcore_barrier()

    def _drain_scatters(k):
        pltpu.make_async_copy(wbuf[k], den_s.at[idxb[k].at[1]], semd[k]).wait()
        pltpu.make_async_copy(rows[k], out_s.at[idxb[k].at[1]], semo[k]).wait()

    def _phase(c, k, has_prev):
        k1 = (k + 1) % 3
        k2 = (k + 2) % 3
        pltpu.make_async_copy(eidx_hbm.at[cid, c + 1], idxb[k1], semi[k1]).wait()
        pltpu.async_copy(hp_hbm.at[idxb[k1].at[0]], rows[k1], semg[k1])
        if has_prev:
            _drain_scatters(k2)
        pltpu.async_copy(eidx_hbm.at[cid, c + 2], idxb[k2], semi[k2])
        for j in range(CH // 16):
            s16 = idxb[k][0, pl.ds(j * 16, 16)]
            d16 = idxb[k][1, pl.ds(j * 16, 16)]
            zz = plsc.load_gather(el_t, [s16]) + plsc.load_gather(er_t, [d16])
            e = jnp.where(zz >= 0, zz, zz * 0.2)
            wbuf[k][pl.ds(j * 16, 16)] = jnp.exp(e - m_vec)
        pltpu.async_copy(wbuf[k], den_s.at[idxb[k].at[1]], semd[k], add=True)
        pltpu.make_async_copy(hp_hbm.at[idxb[k].at[0]], rows[k], semg[k]).wait()
        for j in range(CH // 16):
            w16 = wbuf[k][pl.ds(j * 16, 16)]
            for k2_ in range(16):
                jj = j * 16 + k2_
                rows[k][jj, :] = rows[k][jj, :] * w16[k2_]
        pltpu.async_copy(rows[k], out_s.at[idxb[k].at[1]], semo[k], add=True)

    pltpu.async_copy(eidx_hbm.at[cid, cbase], idxb[0], semi[0])
    pltpu.make_async_copy(eidx_hbm.at[cid, cbase], idxb[0], semi[0]).wait()
    pltpu.async_copy(hp_hbm.at[idxb[0].at[0]], rows[0], semg[0])
    pltpu.async_copy(eidx_hbm.at[cid, cbase + 1], idxb[1], semi[1])
    _phase(cbase + 0, 0, False)
    _phase(cbase + 1, 1, True)
    _phase(cbase + 2, 2, True)
    _phase(cbase + 3, 0, True)

    def _trip(i, _):
        c = cbase + 4 + 3 * i
        _phase(c + 0, 1, True)
        _phase(c + 1, 2, True)
        _phase(c + 2, 0, True)
        return 0
    lax.fori_loop(0, (nch - 4) // 3, _trip, 0)

    kg = nch % 3
    ki = (nch + 1) % 3
    kl = (nch - 1) % 3
    pltpu.make_async_copy(hp_hbm.at[idxb[kg].at[0]], rows[kg], semg[kg]).wait()
    pltpu.make_async_copy(eidx_hbm.at[cid, cbase + nch + 1], idxb[ki],
                          semi[ki]).wait()
    _drain_scatters(kl)

    plsc.subcore_barrier()

    pltpu.sync_copy(out_s.at[pl.ds(row0, NPT), :],
                    s_hbm.at[cid, pl.ds(row0, NPT), :])
    pltpu.sync_copy(den_s.at[pl.ds(row0, NPT)],
                    den_hbm.at[cid, pl.ds(row0, NPT)])


def _make_sc_gat(nch):
    mesh = plsc.VectorSubcoreMesh(core_axis_name="c", subcore_axis_name="s",
                                  num_cores=NC)
    return pl.kernel(
        lambda *args: _sc_gat_body(*args, nch=nch),
        out_type=[
            jax.ShapeDtypeStruct((NC, N_PAD, HID), jnp.float32),
            jax.ShapeDtypeStruct((NC, N_PAD), jnp.float32),
        ],
        mesh=mesh,
        compiler_params=pltpu.CompilerParams(
            needs_layout_passes=False, use_tc_tiling_on_sc=False),
        scratch_types=[
            pltpu.VMEM((2 * N_PAD,), jnp.float32),
            pltpu.VMEM((N_PAD,), jnp.float32),
            pltpu.VMEM((16,), jnp.float32),
            [pltpu.VMEM((2, CH), jnp.int32) for _ in range(3)],
            [pltpu.VMEM((CH,), jnp.float32) for _ in range(3)],
            [pltpu.VMEM((CH, HID), jnp.float32) for _ in range(3)],
            pltpu.VMEM_SHARED((N_PAD, HID), jnp.float32),
            pltpu.VMEM_SHARED((N_PAD,), jnp.float32),
            [pltpu.SemaphoreType.DMA for _ in range(3)],
            [pltpu.SemaphoreType.DMA for _ in range(3)],
            [pltpu.SemaphoreType.DMA for _ in range(3)],
            [pltpu.SemaphoreType.DMA for _ in range(3)],
        ],
    )


def _lstm_body(s_ref, dA_ref, dB_ref, h0_ref, b0_ref, b1_ref,
               wih0_ref, bb0_ref, wih1_ref, whh1_ref, bb1_ref,
               wp_ref, bp_ref, o_ref):
    t0 = s_ref[0] / (dA_ref[...] + EPS)
    t1 = s_ref[1] / (dB_ref[...] + EPS)
    xx = h0_ref[...]
    in0 = jnp.concatenate([t0 + b0_ref[...], xx], axis=-1)
    g = jnp.dot(in0, wih0_ref[...], preferred_element_type=jnp.float32) + bb0_ref[...]
    i, f, gg, o = jnp.split(g, 4, axis=-1)
    i = jax.nn.sigmoid(i)
    gg = jnp.tanh(gg)
    o = jax.nn.sigmoid(o)
    c1 = i * gg
    x1 = o * jnp.tanh(c1)
    in1 = jnp.concatenate([t1 + b1_ref[...], x1], axis=-1)
    g = (jnp.dot(in1, wih1_ref[...], preferred_element_type=jnp.float32)
         + jnp.dot(x1, whh1_ref[...], preferred_element_type=jnp.float32)
         + bb1_ref[...])
    i, f, gg, o = jnp.split(g, 4, axis=-1)
    i = jax.nn.sigmoid(i)
    f = jax.nn.sigmoid(f)
    gg = jnp.tanh(gg)
    o = jax.nn.sigmoid(o)
    c2 = f * c1 + i * gg
    x2 = o * jnp.tanh(c2)
    y = jnp.dot(x2, wp_ref[...], preferred_element_type=jnp.float32) + bp_ref[...]
    o_ref[...] = jax.nn.sigmoid(y)


def _tc_lstm(s2, dA2, dB2, h0, b0, b1, W_ih0, b_ih0, b_hh0,
             W_ih1, W_hh1, b_ih1, b_hh1, W_pred, b_pred):
    wp128 = jnp.zeros((HID, 128), jnp.float32).at[:, 0].set(W_pred[:, 0])
    bp128 = jnp.zeros((1, 128), jnp.float32).at[0, 0].set(b_pred[0])
    blk = 2048
    node = lambda w: pl.BlockSpec((blk, w), lambda i: (i, 0))
    s_spec = pl.BlockSpec((2, blk, HID), lambda i: (0, i, 0))
    full = lambda a: pl.BlockSpec(a.shape, lambda i: (0,) * a.ndim)
    args = (s2, dA2, dB2, h0,
            b0.reshape(1, -1), b1.reshape(1, -1),
            W_ih0.T, (b_ih0 + b_hh0).reshape(1, -1),
            W_ih1.T, W_hh1.T, (b_ih1 + b_hh1).reshape(1, -1),
            wp128, bp128)
    specs = [s_spec, node(1), node(1), node(16)]
    specs += [full(a) for a in args[4:]]
    return pl.pallas_call(
        _lstm_body,
        grid=(N_PAD // blk,),
        in_specs=specs,
        out_specs=pl.BlockSpec((blk, 128), lambda i: (i, 0)),
        out_shape=jax.ShapeDtypeStruct((N_PAD, 128), jnp.float32),
    )(*args)


def kernel(x, edge_index, W_embed, b_embed, W_lin1, b_lin1,
           W_gat0, attn_l0, attn_r0, b_gat0,
           W_gat1, attn_l1, attn_r1, b_gat1,
           W_ih0, W_hh0, b_ih0, b_hh0,
           W_ih1, W_hh1, b_ih1, b_hh1,
           W_pred, b_pred):
    n, _ = x.shape
    e = edge_index.shape[1]
    grp = NS * CH
    nch = (e + grp - 1) // grp
    while nch % 3 != 1 or nch < 7:
        nch += 1
    e_pad = nch * grp

    x_p = jnp.concatenate(
        [x, jnp.zeros((N_PAD - n, x.shape[1]), jnp.float32)], axis=0)
    pad_idx = jnp.full((e_pad - e,), N_PAD - 1, jnp.int32)
    src_p = jnp.concatenate([edge_index[0], pad_idx])
    dst_p = jnp.concatenate([edge_index[1], pad_idx])
    eidx = jnp.stack([src_p.reshape(-1, CH), dst_p.reshape(-1, CH)], axis=1)
    eidx = jnp.concatenate(
        [eidx, jnp.full((2, 2, CH), N_PAD - 1, jnp.int32)], axis=0)
    eidx_b = jnp.concatenate([eidx[:, :1, :] + N_PAD, eidx[:, 1:, :]], axis=1)
    eidx2 = jnp.stack([eidx, eidx_b], axis=0)

    sc_gat = _make_sc_gat(nch)

    h0, hp1, eler1 = _tc_embed(x_p, W_embed, b_embed, W_lin1, b_lin1,
                               W_gat0, attn_l0, attn_r0,
                               W_gat1, attn_l1, attn_r1)
    s1, den1 = sc_gat(eidx2, hp1, eler1)
    dA1 = den1[0].reshape(N_PAD, 1)
    dB1 = den1[1].reshape(N_PAD, 1)
    hp2, eler2 = _tc_prep2(s1, dA1, dB1, b_gat0, b_gat1,
                           W_gat0, attn_l0, attn_r0,
                           W_gat1, attn_l1, attn_r1)
    s2, den2 = sc_gat(eidx2, hp2, eler2)
    dA2 = den2[0].reshape(N_PAD, 1)
    dB2 = den2[1].reshape(N_PAD, 1)

    y = _tc_lstm(s2, dA2, dB2, h0, b_gat0, b_gat1,
                 W_ih0, b_ih0, b_hh0, W_ih1, W_hh1, b_ih1, b_hh1,
                 W_pred, b_pred)
    return y[:n, :1]
